# baseline (speedup 1.0000x reference)
import functools

import jax
import jax.numpy as jnp
from jax import lax
from jax.experimental import pallas as pl
from jax.experimental.pallas import tpu as pltpu
from jax.experimental.pallas import tpu_sc as plsc

NUM_USERS = 1000000
NUM_SONGS = 1000000
EMBED = 32
BATCH = 16384

_INFO = plsc.get_sparse_core_info()
NC = _INFO.num_cores
NS = _INFO.num_subcores
L = _INFO.num_lanes
NW = NC * NS
B_PER_W = BATCH // NW
CHUNK = 128
NCHUNK = B_PER_W // CHUNK
NBLK = B_PER_W // L


def _body(uid_hbm, sid_hbm, utab_hbm, stab_hbm, out_hbm,
          uidx_v, sidx_v, urows_v, srows_v, out_v, sem):
    wid = lax.axis_index("s") * NC + lax.axis_index("c")
    base = wid * B_PER_W

    pltpu.sync_copy(uid_hbm.at[wid], uidx_v)
    pltpu.sync_copy(sid_hbm.at[wid], sidx_v)

    copies = []
    for j in range(NCHUNK):
        copies.append(pltpu.async_copy(
            utab_hbm.at[uidx_v.at[j]], urows_v.at[pl.ds(j * CHUNK, CHUNK)],
            sem))
        copies.append(pltpu.async_copy(
            stab_hbm.at[sidx_v.at[j]], srows_v.at[pl.ds(j * CHUNK, CHUNK)],
            sem))
    for c in copies:
        c.wait()

    lane = lax.iota(jnp.int32, L)

    def blk(b, carry):
        rows = b * L + lane
        acc = jnp.zeros((L,), jnp.float32)
        for d in range(EMBED):
            dd = jnp.full((L,), d, jnp.int32)
            gu = plsc.load_gather(urows_v, [rows, dd])
            gs = plsc.load_gather(srows_v, [rows, dd])
            acc = acc + gu * gs
        rating = 10.0 / (1.0 + jnp.exp(-acc))
        out_v[pl.ds(b * L, L)] = rating
        return carry

    lax.fori_loop(0, NBLK, blk, 0)

    pltpu.sync_copy(out_v, out_hbm.at[pl.ds(base, B_PER_W)])


@jax.jit
def kernel(user_id, song_id, user_table, song_table):
    uid = user_id.astype(jnp.int32).reshape(NW, NCHUNK, CHUNK)
    sid = song_id.astype(jnp.int32).reshape(NW, NCHUNK, CHUNK)
    mesh = plsc.VectorSubcoreMesh(core_axis_name="c", subcore_axis_name="s")
    run = pl.kernel(
        _body,
        mesh=mesh,
        out_type=jax.ShapeDtypeStruct((BATCH,), jnp.float32),
        scratch_types=[
            pltpu.VMEM((NCHUNK, CHUNK), jnp.int32),
            pltpu.VMEM((NCHUNK, CHUNK), jnp.int32),
            pltpu.VMEM((B_PER_W, EMBED), jnp.float32),
            pltpu.VMEM((B_PER_W, EMBED), jnp.float32),
            pltpu.VMEM((B_PER_W,), jnp.float32),
            pltpu.SemaphoreType.DMA,
        ],
        compiler_params=pltpu.CompilerParams(
            needs_layout_passes=False, use_tc_tiling_on_sc=False),
    )
    return run(uid, sid, user_table, song_table)


# speedup vs baseline: 4.3150x; 4.3150x over previous
import jax
import jax.numpy as jnp
from jax import lax
from jax.experimental import pallas as pl
from jax.experimental.pallas import tpu as pltpu
from jax.experimental.pallas import tpu_sc as plsc

EMBED = 32
BATCH = 16384
NUSERS = 1000000

_INFO = plsc.get_sparse_core_info()
NC = _INFO.num_cores
NS = _INFO.num_subcores
L = _INFO.num_lanes
NW = NC * NS
B_PER_W = BATCH // NW
RING = 8


def _body(uid_hbm, sid_hbm, utab_hbm, stab_hbm, out_hbm,
          uid_v, sid_v, uring_v, sring_v, part_v, out_v, sems):
    wid = lax.axis_index("s") * NC + lax.axis_index("c")
    base = wid * B_PER_W

    for j in range(B_PER_W // 128):
        pltpu.sync_copy(uid_hbm.at[pl.ds(base + j * 128, 128)],
                        uid_v.at[pl.ds(j * 128, 128)])
        pltpu.sync_copy(sid_hbm.at[pl.ds(base + j * 128, 128)],
                        sid_v.at[pl.ds(j * 128, 128)])

    def fire(e, slot):
        u = uid_v[pl.ds(e, L)][0]
        s = sid_v[pl.ds(e, L)][0]
        ustart = pl.multiple_of(lax.shift_right_logical(u, 7) * 128, 128)
        sstart = pl.multiple_of(lax.shift_right_logical(s, 7) * 128, 128)
        pltpu.async_copy(utab_hbm.at[:, :, pl.ds(ustart, 128)],
                         uring_v.at[slot], sems.at[slot])
        pltpu.async_copy(stab_hbm.at[:, :, pl.ds(sstart, 128)],
                         sring_v.at[slot], sems.at[slot])

    lane = lax.iota(jnp.int32, L)
    rt_lo = lax.shift_right_logical(lane, 3)
    rt_hi = rt_lo + 2
    sub = lane & 7

    for e in range(RING):
        fire(e, e)

    def step(e, carry):
        slot = e & (RING - 1)
        pltpu.make_async_copy(utab_hbm.at[:, :, pl.ds(0, 128)],
                              uring_v.at[slot], sems.at[slot]).wait()
        pltpu.make_async_copy(stab_hbm.at[:, :, pl.ds(0, 128)],
                              sring_v.at[slot], sems.at[slot]).wait()
        ucol = jnp.full((L,), uid_v[pl.ds(e, L)][0] & 127, jnp.int32)
        scol = jnp.full((L,), sid_v[pl.ds(e, L)][0] & 127, jnp.int32)
        ub = uring_v.at[slot]
        sb = sring_v.at[slot]
        gu_lo = plsc.load_gather(ub, [rt_lo, sub, ucol])
        gu_hi = plsc.load_gather(ub, [rt_hi, sub, ucol])
        gs_lo = plsc.load_gather(sb, [rt_lo, sub, scol])
        gs_hi = plsc.load_gather(sb, [rt_hi, sub, scol])
        part_v[pl.ds(e * L, L)] = gu_lo * gs_lo + gu_hi * gs_hi

        @pl.when(e + RING < B_PER_W)
        def _():
            fire(e + RING, slot)

        return carry

    lax.fori_loop(0, B_PER_W, step, 0)

    def blk(b, carry):
        ev = (b * L + lane) * L
        acc = jnp.zeros((L,), jnp.float32)
        for l in range(L):
            acc = acc + plsc.load_gather(part_v, [ev + l])
        rating = 10.0 / (1.0 + jnp.exp(-acc))
        out_v[b >> 3, pl.ds((b & 7) * L, L)] = rating
        return carry

    lax.fori_loop(0, B_PER_W // L, blk, 0)

    for j in range(B_PER_W // 128):
        pltpu.sync_copy(out_v.at[j],
                        out_hbm.at[pl.ds(base + j * 128, 128)])


@jax.jit
def kernel(user_id, song_id, user_table, song_table):
    uid = user_id.astype(jnp.int32)
    sid = song_id.astype(jnp.int32)
    utab = user_table.T.reshape(4, 8, NUSERS)
    stab = song_table.T.reshape(4, 8, NUSERS)
    mesh = plsc.VectorSubcoreMesh(core_axis_name="c", subcore_axis_name="s")
    run = pl.kernel(
        _body,
        mesh=mesh,
        out_type=jax.ShapeDtypeStruct((BATCH,), jnp.float32),
        scratch_types=[
            pltpu.VMEM((B_PER_W + L,), jnp.int32),
            pltpu.VMEM((B_PER_W + L,), jnp.int32),
            pltpu.VMEM((RING, 4, 8, 128), jnp.float32),
            pltpu.VMEM((RING, 4, 8, 128), jnp.float32),
            pltpu.VMEM((B_PER_W * L,), jnp.float32),
            pltpu.VMEM((B_PER_W // 128, 128), jnp.float32),
            pltpu.SemaphoreType.DMA((RING,)),
        ],
        compiler_params=pltpu.CompilerParams(
            needs_layout_passes=False, use_tc_tiling_on_sc=True),
    )
    return run(uid, sid, utab, stab)
